# Initial kernel scaffold; baseline (speedup 1.0000x reference)
import jax
import jax.numpy as jnp
from jax.experimental import pallas as pl


def kernel(inputs, embeddings):
    raise NotImplementedError("write your pallas kernel here")


# speedup vs baseline: 1.1099x; 1.1099x over previous
import functools

import jax
import jax.numpy as jnp
from jax import lax
from jax.experimental import pallas as pl
from jax.experimental.pallas import tpu as pltpu
from jax.experimental.pallas import tpu_sc as plsc

DIM = 32
NUM_CORES = 2
NUM_SUBCORES = 16
NUM_WORKERS = NUM_CORES * NUM_SUBCORES


@functools.partial(jax.jit, static_argnames=("chunk", "n_steps"))
def _gather_sc(flat_ids, embeddings, chunk, n_steps):
    B = flat_ids.shape[0]
    mesh = plsc.VectorSubcoreMesh(
        core_axis_name="c",
        subcore_axis_name="s",
        num_cores=NUM_CORES,
        num_subcores=NUM_SUBCORES,
    )

    @functools.partial(
        pl.kernel,
        mesh=mesh,
        compiler_params=pltpu.CompilerParams(use_tc_tiling_on_sc=False),
        out_type=jax.ShapeDtypeStruct((B, DIM), jnp.float32),
        scratch_types=[
            pltpu.VMEM((chunk,), jnp.int32),
            pltpu.VMEM((chunk, DIM), jnp.float32),
            pltpu.SemaphoreType.DMA,
        ],
    )
    def k(idx_hbm, table_hbm, out_hbm, idx_v, rows_v, sem):
        wid = lax.axis_index("s") * NUM_CORES + lax.axis_index("c")
        base = wid * (chunk * n_steps)

        def body(i, carry):
            off = base + i * chunk
            pltpu.sync_copy(idx_hbm.at[pl.ds(off, chunk)], idx_v)
            pltpu.async_copy(table_hbm.at[idx_v], rows_v, sem).wait()
            pltpu.sync_copy(rows_v, out_hbm.at[pl.ds(off, chunk)])
            return carry

        lax.fori_loop(0, n_steps, body, 0)

    return k(flat_ids, embeddings)


def kernel(inputs, embeddings):
    shape = inputs.shape
    flat = jnp.reshape(inputs, (-1,)).astype(jnp.int32)
    B = flat.shape[0]
    b_per_w = B // NUM_WORKERS
    chunk = 3200
    n_steps = b_per_w // chunk
    out = _gather_sc(flat, embeddings, chunk, n_steps)
    return jnp.reshape(out, shape + (DIM,))


# speedup vs baseline: 1.1120x; 1.0019x over previous
import functools

import jax
import jax.numpy as jnp
from jax import lax
from jax.experimental import pallas as pl
from jax.experimental.pallas import tpu as pltpu
from jax.experimental.pallas import tpu_sc as plsc

DIM = 32
NUM_CORES = 2
NUM_SUBCORES = 16
NUM_WORKERS = NUM_CORES * NUM_SUBCORES


@functools.partial(jax.jit, static_argnames=("chunk", "n_steps"))
def _gather_sc(flat_ids, embeddings, chunk, n_steps):
    B = flat_ids.shape[0]
    mesh = plsc.VectorSubcoreMesh(
        core_axis_name="c",
        subcore_axis_name="s",
        num_cores=NUM_CORES,
        num_subcores=NUM_SUBCORES,
    )

    @functools.partial(
        pl.kernel,
        mesh=mesh,
        compiler_params=pltpu.CompilerParams(use_tc_tiling_on_sc=False),
        out_type=jax.ShapeDtypeStruct((B, DIM), jnp.float32),
        scratch_types=[
            pltpu.VMEM((chunk,), jnp.int32),
            pltpu.VMEM((chunk,), jnp.int32),
            pltpu.VMEM((chunk, DIM), jnp.float32),
            pltpu.VMEM((chunk, DIM), jnp.float32),
            pltpu.SemaphoreType.DMA,
            pltpu.SemaphoreType.DMA,
            pltpu.SemaphoreType.DMA,
            pltpu.SemaphoreType.DMA,
        ],
    )
    def k(idx_hbm, table_hbm, out_hbm, idx0, idx1, rows0, rows1,
          gsem0, gsem1, osem0, osem1):
        idx_b = (idx0, idx1)
        rows_b = (rows0, rows1)
        gsem_b = (gsem0, gsem1)
        osem_b = (osem0, osem1)

        wid = lax.axis_index("s") * NUM_CORES + lax.axis_index("c")
        base = wid * (chunk * n_steps)

        for b in range(2):
            pltpu.sync_copy(idx_hbm.at[pl.ds(base + b * chunk, chunk)],
                            idx_b[b])
            pltpu.async_copy(table_hbm.at[idx_b[b]], rows_b[b], gsem_b[b])

        def body(o, carry):
            for b in range(2):
                i = 2 * o + b
                off = base + i * chunk
                pltpu.make_async_copy(
                    table_hbm.at[idx_b[b]], rows_b[b], gsem_b[b]).wait()
                pltpu.async_copy(
                    rows_b[b], out_hbm.at[pl.ds(off, chunk)], osem_b[b])
                pltpu.sync_copy(
                    idx_hbm.at[pl.ds(off + 2 * chunk, chunk)], idx_b[b])
                pltpu.make_async_copy(
                    rows_b[b], out_hbm.at[pl.ds(off, chunk)],
                    osem_b[b]).wait()
                pltpu.async_copy(table_hbm.at[idx_b[b]], rows_b[b], gsem_b[b])
            return carry

        lax.fori_loop(0, n_steps // 2 - 1, body, 0)

        for b in range(2):
            i = n_steps - 2 + b
            off = base + i * chunk
            pltpu.make_async_copy(
                table_hbm.at[idx_b[b]], rows_b[b], gsem_b[b]).wait()
            pltpu.async_copy(
                rows_b[b], out_hbm.at[pl.ds(off, chunk)], osem_b[b])
        for b in range(2):
            i = n_steps - 2 + b
            off = base + i * chunk
            pltpu.make_async_copy(
                rows_b[b], out_hbm.at[pl.ds(off, chunk)], osem_b[b]).wait()

    return k(flat_ids, embeddings)


def kernel(inputs, embeddings):
    shape = inputs.shape
    flat = jnp.reshape(inputs, (-1,)).astype(jnp.int32)
    B = flat.shape[0]
    b_per_w = B // NUM_WORKERS
    chunk = 1600
    n_steps = b_per_w // chunk
    out = _gather_sc(flat, embeddings, chunk, n_steps)
    return jnp.reshape(out, shape + (DIM,))


# speedup vs baseline: 1.6048x; 1.4431x over previous
import functools

import jax
import jax.numpy as jnp
from jax import lax
from jax.experimental import pallas as pl
from jax.experimental.pallas import tpu as pltpu
from jax.experimental.pallas import tpu_sc as plsc

DIM = 32
NUM_CORES = 2
NUM_SUBCORES = 16
NUM_WORKERS = NUM_CORES * NUM_SUBCORES
RB = 4
S = RB * 128


@jax.jit
def _gather_sc(ids_l, embeddings):
    B = ids_l.shape[0]
    R = 16384
    NL = B // R
    n_units = NL * (R // (RB * 128))
    U = n_units // NUM_WORKERS
    n_rh = R // 128
    mesh = plsc.VectorSubcoreMesh(
        core_axis_name="c",
        subcore_axis_name="s",
        num_cores=NUM_CORES,
        num_subcores=NUM_SUBCORES,
    )

    @functools.partial(
        pl.kernel,
        mesh=mesh,
        compiler_params=pltpu.CompilerParams(
            use_tc_tiling_on_sc=False, needs_layout_passes=False),
        out_type=jax.ShapeDtypeStruct((NL * 4 * n_rh * 8 * 128,), jnp.float32),
        scratch_types=[
            pltpu.VMEM((S,), jnp.int32),
            pltpu.VMEM((S,), jnp.int32),
            pltpu.VMEM((S, DIM), jnp.float32),
            pltpu.VMEM((S, DIM), jnp.float32),
            pltpu.VMEM((4 * RB * 8 * 128,), jnp.float32),
            pltpu.VMEM((4 * RB * 8 * 128,), jnp.float32),
            pltpu.SemaphoreType.DMA,
            pltpu.SemaphoreType.DMA,
            pltpu.SemaphoreType.DMA,
            pltpu.SemaphoreType.DMA,
        ],
    )
    def k(idx_hbm, table_hbm, out_hbm, idx0, idx1, rows0, rows1,
          t0, t1, gsem0, gsem1, osem0, osem1):
        idx_b = (idx0, idx1)
        rows_b = (rows0, rows1)
        t_b = (t0, t1)
        gsem_b = (gsem0, gsem1)
        osem_b = (osem0, osem1)

        wid = lax.axis_index("s") * NUM_CORES + lax.axis_index("c")
        u0 = wid * U
        iota = lax.iota(jnp.int32, 16)

        def idx_off(u):
            l = u // (n_rh // RB)
            rhb = u % (n_rh // RB)
            return l * R + rhb * S

        def out_off(u, ch):
            l = u // (n_rh // RB)
            rhb = u % (n_rh // RB)
            return ((l * 4 + ch) * n_rh + rhb * RB) * 1024

        def fire(u, b):
            pltpu.sync_copy(idx_hbm.at[pl.ds(idx_off(u), S)], idx_b[b])
            pltpu.async_copy(table_hbm.at[idx_b[b]], rows_b[b], gsem_b[b])

        def transpose_unit(b):
            rows, t = rows_b[b], t_b[b]

            def vbody(v, carry):
                sidx = iota + v * 16
                obase = (v // 8) * 1024 + (v % 8) * 16
                for c in range(DIM):
                    vec = plsc.load_gather(
                        rows, [sidx, jnp.full((16,), c, jnp.int32)])
                    off = obase + (c // 8) * 4096 + (c % 8) * 128
                    t[pl.ds(off, 16)] = vec
                return carry

            lax.fori_loop(0, S // 16, vbody, 0)

        def store_unit(u, b):
            for ch in range(4):
                pltpu.async_copy(
                    t_b[b].at[pl.ds(ch * RB * 1024, RB * 1024)],
                    out_hbm.at[pl.ds(out_off(u, ch), RB * 1024)],
                    osem_b[b])

        def drain_unit(u, b):
            for ch in range(4):
                pltpu.make_async_copy(
                    t_b[b].at[pl.ds(ch * RB * 1024, RB * 1024)],
                    out_hbm.at[pl.ds(out_off(u, ch), RB * 1024)],
                    osem_b[b]).wait()

        fire(u0, 0)

        def body(o, carry):
            for b in range(2):
                i = 2 * o + b
                u = u0 + i

                @pl.when(i + 1 < U)
                def _():
                    fire(u + 1, 1 - b)

                pltpu.make_async_copy(
                    table_hbm.at[idx_b[b]], rows_b[b], gsem_b[b]).wait()

                @pl.when(i >= 2)
                def _():
                    drain_unit(u - 2, b)

                transpose_unit(b)
                store_unit(u, b)
            return carry

        lax.fori_loop(0, U // 2, body, 0)

        for b in range(2):
            drain_unit(u0 + U - 2 + b, b)

    return k(ids_l, embeddings)


def kernel(inputs, embeddings):
    R, NL = inputs.shape
    ids_l = jnp.reshape(jnp.transpose(inputs), (-1,)).astype(jnp.int32)
    out5 = _gather_sc(ids_l, embeddings)
    out = jnp.reshape(out5, (NL, 4, R // 128, 8, 128))
    out = jnp.transpose(out, (2, 4, 0, 1, 3))
    return jnp.reshape(out, (R, NL, DIM))


# speedup vs baseline: 1.9657x; 1.2249x over previous
import functools

import jax
import jax.numpy as jnp
from jax import lax
from jax.experimental import pallas as pl
from jax.experimental.pallas import tpu as pltpu
from jax.experimental.pallas import tpu_sc as plsc

DIM = 32
NUM_CORES = 2
NUM_SUBCORES = 16
NUM_WORKERS = NUM_CORES * NUM_SUBCORES
RB = 4
S = RB * 128


@jax.jit
def _gather_sc(ids_l, embeddings):
    B = ids_l.shape[0]
    R = 16384
    NL = B // R
    n_units = NL * (R // (RB * 128))
    U = n_units // NUM_WORKERS
    n_rh = R // 128
    mesh = plsc.VectorSubcoreMesh(
        core_axis_name="c",
        subcore_axis_name="s",
        num_cores=NUM_CORES,
        num_subcores=NUM_SUBCORES,
    )

    @functools.partial(
        pl.kernel,
        mesh=mesh,
        compiler_params=pltpu.CompilerParams(
            use_tc_tiling_on_sc=False, needs_layout_passes=False,
            disable_bounds_checks=True),
        out_type=jax.ShapeDtypeStruct((NL * 4 * n_rh * 8 * 128,), jnp.float32),
        scratch_types=[
            pltpu.VMEM((S,), jnp.int32),
            pltpu.VMEM((S,), jnp.int32),
            pltpu.VMEM((S, DIM), jnp.float32),
            pltpu.VMEM((S, DIM), jnp.float32),
            pltpu.VMEM((4 * RB * 8 * 128,), jnp.float32),
            pltpu.VMEM((4 * RB * 8 * 128,), jnp.float32),
            pltpu.SemaphoreType.DMA,
            pltpu.SemaphoreType.DMA,
            pltpu.SemaphoreType.DMA,
            pltpu.SemaphoreType.DMA,
        ],
    )
    def k(idx_hbm, table_hbm, out_hbm, idx0, idx1, rows0, rows1,
          t0, t1, gsem0, gsem1, osem0, osem1):
        idx_b = (idx0, idx1)
        rows_b = (rows0, rows1)
        t_b = (t0, t1)
        gsem_b = (gsem0, gsem1)
        osem_b = (osem0, osem1)

        wid = lax.axis_index("s") * NUM_CORES + lax.axis_index("c")
        u0 = wid * U
        iota = lax.iota(jnp.int32, 16)

        def idx_off(u):
            l = u // (n_rh // RB)
            rhb = u % (n_rh // RB)
            return l * R + rhb * S

        def out_off(u, ch):
            l = u // (n_rh // RB)
            rhb = u % (n_rh // RB)
            return ((l * 4 + ch) * n_rh + rhb * RB) * 1024

        def fire(u, b):
            pltpu.sync_copy(idx_hbm.at[pl.ds(idx_off(u), S)], idx_b[b])
            pltpu.async_copy(table_hbm.at[idx_b[b]], rows_b[b], gsem_b[b])

        def transpose_unit(b):
            rows, t = rows_b[b], t_b[b]

            @plsc.parallel_loop(0, S // 16, unroll=4)
            def vbody(v):
                sidx = iota + v * 16
                obase = (v // 8) * 1024 + (v % 8) * 16
                for c in range(DIM):
                    vec = plsc.load_gather(
                        rows, [sidx, jnp.full((16,), c, jnp.int32)])
                    off = obase + (c // 8) * 4096 + (c % 8) * 128
                    t[pl.ds(off, 16)] = vec

        def store_unit(u, b):
            for ch in range(4):
                pltpu.async_copy(
                    t_b[b].at[pl.ds(ch * RB * 1024, RB * 1024)],
                    out_hbm.at[pl.ds(out_off(u, ch), RB * 1024)],
                    osem_b[b])

        def drain_unit(u, b):
            for ch in range(4):
                pltpu.make_async_copy(
                    t_b[b].at[pl.ds(ch * RB * 1024, RB * 1024)],
                    out_hbm.at[pl.ds(out_off(u, ch), RB * 1024)],
                    osem_b[b]).wait()

        fire(u0, 0)

        def body(o, carry):
            for b in range(2):
                i = 2 * o + b
                u = u0 + i

                @pl.when(i + 1 < U)
                def _():
                    fire(u + 1, 1 - b)

                pltpu.make_async_copy(
                    table_hbm.at[idx_b[b]], rows_b[b], gsem_b[b]).wait()

                @pl.when(i >= 2)
                def _():
                    drain_unit(u - 2, b)

                transpose_unit(b)
                store_unit(u, b)
            return carry

        lax.fori_loop(0, U // 2, body, 0)

        for b in range(2):
            drain_unit(u0 + U - 2 + b, b)

    return k(ids_l, embeddings)


def kernel(inputs, embeddings):
    R, NL = inputs.shape
    ids_l = jnp.reshape(jnp.transpose(inputs), (-1,)).astype(jnp.int32)
    out5 = _gather_sc(ids_l, embeddings)
    out = jnp.reshape(out5, (NL, 4, R // 128, 8, 128))
    out = jnp.transpose(out, (2, 4, 0, 1, 3))
    return jnp.reshape(out, (R, NL, DIM))
